# baseline (speedup 1.0000x reference)
import jax
import jax.numpy as jnp
from jax.experimental import pallas as pl
from jax.experimental.pallas import tpu as pltpu

_THRESHOLD = 0.95
_MV_BLK = 1024
_NSUB = 4
_NL = 1024
_S = _NSUB * _NL
_B = 4
_ROWS = _B * _NSUB


def _roll(x, shift, axis):
    return jnp.roll(x, shift, axis)


def _sub_levels():
    m, out = 1, []
    while m < _NSUB:
        out.append(m)
        m *= 2
    return tuple(out)


def _spread_from_c0(v, csub):
    for m in _sub_levels():
        v = jnp.where((csub & m) == 0, v, _roll(v, m, 0))
    return v


def _lane_fold(t):
    w = t.shape[1]
    while w > 1:
        w //= 2
        t = t[:, :w] + t[:, w:2 * w]
    return t


def _group_reduce_sum_exact(x, csub):
    s = jnp.sum(x, axis=1, keepdims=True)
    for m in _sub_levels():
        s = s + _roll(s, -m, 0)
    return _spread_from_c0(s, csub)


def _group_reduce_max(x, csub):
    s = jnp.max(x, axis=1, keepdims=True)
    for m in _sub_levels():
        s = jnp.maximum(s, _roll(s, -m, 0))
    return _spread_from_c0(s, csub)


def _row_total(gated, csub):
    nl = gated.shape[1]
    q = nl // 4
    u = (gated[:, :q] + gated[:, q:2 * q]) + \
        (gated[:, 2 * q:3 * q] + gated[:, 3 * q:])
    for m in _sub_levels():
        u = u + _roll(u, -m, 0)
    t = _lane_fold(u)
    return _spread_from_c0(t, csub[:, :1])


def _bitonic_desc(x, lane, csub):
    rows, nl = x.shape

    def gbit_zero(t):
        if (1 << t) < nl:
            return (lane & (1 << t)) == 0
        return (csub & ((1 << t) // nl)) == 0

    k, tk = 2, 1
    while k <= _S:
        dir_desc = gbit_zero(tk) if k < _S else jnp.bool_(True)
        j, tj = k // 2, tk - 1
        while j >= 1:
            is_lower = gbit_zero(tj)
            if j < nl:
                partner = jnp.where(is_lower, _roll(x, -j, 1), _roll(x, j, 1))
            else:
                m = j // nl
                partner = jnp.where(is_lower, _roll(x, -m, 0), _roll(x, m, 0))
            mx = jnp.maximum(x, partner)
            mn = jnp.minimum(x, partner)
            take_max = jnp.logical_not(jnp.logical_xor(dir_desc, is_lower))
            x = jnp.where(take_max, mx, mn)
            j //= 2
            tj -= 1
        k *= 2
        tk += 1
    return x


def _build_mask_row(lg, am, gates_out_ref, keep_out_ref, roff,
                    xt_ref, cumt_ref):
    rows, nl = lg.shape
    lane = jax.lax.broadcasted_iota(jnp.int32, (rows, nl), 1)
    rowi = jax.lax.broadcasted_iota(jnp.int32, (rows, nl), 0)
    csub = rowi & (_NSUB - 1)
    glob = csub * nl + lane

    gates = jax.nn.sigmoid(lg)
    gates_out_ref[roff:roff + _NSUB, :] = gates
    act = am != 0
    gated = jnp.where(act, gates, jnp.float32(0.0))

    total = jnp.maximum(_row_total(gated, csub), jnp.float32(1e-12))
    shares = jnp.where(act, gated / total, jnp.float32(0.0))

    srt = _bitonic_desc(shares, lane, csub)

    xt_ref[...] = srt.reshape(rows * (nl // 128), 128).T

    def body(i, acc):
        acc = acc + xt_ref[pl.ds(i, 1), :]
        cumt_ref[pl.ds(i, 1), :] = acc
        return acc

    ncols = rows * (nl // 128)
    tot = jax.lax.fori_loop(0, 128, body,
                            jnp.zeros((1, ncols), jnp.float32))

    clane = jax.lax.broadcasted_iota(jnp.int32, (1, ncols), 1)
    s = tot
    for step in range(1, ncols):
        s = jnp.where(clane == step, s + _roll(s, 1, 1), s)
    ex = jnp.where(clane == 0, jnp.float32(0.0), _roll(s, 1, 1))

    cum = (cumt_ref[...] + ex).T.reshape(rows, nl)

    k0 = _group_reduce_sum_exact(
        (cum <= jnp.float32(_THRESHOLD)).astype(jnp.float32), csub)
    kk = jnp.maximum(k0, jnp.float32(1.0))

    sel = glob == (kk.astype(jnp.int32) - 1)
    sstar = _group_reduce_max(jnp.where(sel, srt, jnp.float32(-1.0)), csub)
    n_greater = _group_reduce_sum_exact(
        (srt > sstar).astype(jnp.float32), csub)
    m = kk - n_greater

    eq = shares == sstar
    p = eq.astype(jnp.float32)
    d = 1
    while d < _S:
        if d < nl:
            piece1 = jnp.where(lane >= d, _roll(p, d, 1), jnp.float32(0.0))
            piece2 = jnp.where((lane < d) & (csub > 0),
                               _roll(_roll(p, -(nl - d), 1), 1, 0),
                               jnp.float32(0.0))
            p = p + piece1 + piece2
        else:
            ms = d // nl
            p = p + jnp.where(csub >= ms, _roll(p, ms, 0), jnp.float32(0.0))
        d *= 2

    keep = act & ((shares > sstar) | (eq & (p <= m)))
    keep_out_ref[roff:roff + _NSUB, :] = keep.astype(jnp.int32)


def _fused_kernel(h_ref, w_ref, b_ref, am_ref,
                  logits_ref, gates_ref, keep_ref,
                  lg_ref, xt_ref, cumt_ref):
    i = pl.program_id(0)
    nsteps = pl.num_programs(0)
    spr = _S // _MV_BLK

    lg = jax.lax.dot_general(
        w_ref[...], h_ref[...], (((1,), (1,)), ((), ())),
        preferred_element_type=jnp.float32) + b_ref[...]
    logits_ref[...] = lg[None]
    lg_ref[pl.ds(i, 1), :] = lg

    for r in range(_B):
        step_r = spr * (r + 1) if r < _B - 1 else nsteps - 1
        roff = r * _NSUB

        @pl.when(i == step_r)
        def _(roff=roff):
            lg4 = lg_ref[roff:roff + _NSUB, :]
            am4 = am_ref[roff:roff + _NSUB, :]
            _build_mask_row(lg4, am4, gates_ref, keep_ref, roff,
                            xt_ref, cumt_ref)


def kernel(hidden_states, attention_mask, W, b):
    bb, s, d = hidden_states.shape
    h2d = hidden_states.reshape(bb * s, d)
    rows = bb * _NSUB
    nsteps = bb * s // _MV_BLK

    logits_p, gates_p, keep_p = pl.pallas_call(
        _fused_kernel,
        grid=(nsteps,),
        in_specs=[pl.BlockSpec((_MV_BLK, d), lambda i: (i, 0)),
                  pl.BlockSpec((1, d), lambda i: (0, 0)),
                  pl.BlockSpec((1, 1), lambda i: (0, 0)),
                  pl.BlockSpec((rows, _NL), lambda i: (0, 0))],
        out_specs=[pl.BlockSpec((1, 1, _NL), lambda i: (i, 0, 0)),
                   pl.BlockSpec((rows, _NL), lambda i: (0, 0)),
                   pl.BlockSpec((rows, _NL), lambda i: (0, 0))],
        out_shape=[jax.ShapeDtypeStruct((nsteps, 1, _NL), jnp.float32),
                   jax.ShapeDtypeStruct((rows, _NL), jnp.float32),
                   jax.ShapeDtypeStruct((rows, _NL), jnp.int32)],
        scratch_shapes=[pltpu.VMEM((rows, _NL), jnp.float32),
                        pltpu.VMEM((128, _NSUB * _NL // 128), jnp.float32),
                        pltpu.VMEM((128, _NSUB * _NL // 128), jnp.float32)],
    )(h2d, W.reshape(1, d),
      jnp.asarray(b, jnp.float32).reshape(1, 1),
      attention_mask.reshape(rows, _NL))

    return (logits_p.reshape(bb, s), gates_p.reshape(bb, s),
            keep_p.reshape(bb, s).astype(jnp.bool_))


# speedup vs baseline: 1.4763x; 1.4763x over previous
import jax
import jax.numpy as jnp
from jax.experimental import pallas as pl
from jax.experimental.pallas import tpu as pltpu

_THRESHOLD = 0.95
_MV_BLK = 1024
_NSUB = 4
_NL = 1024
_S = _NSUB * _NL
_B = 4
_ROWS = _B * _NSUB


def _roll(x, shift, axis):
    return jnp.roll(x, shift, axis)


def _sub_levels():
    m, out = 1, []
    while m < _NSUB:
        out.append(m)
        m *= 2
    return tuple(out)


def _spread_from_c0(v, csub):
    for m in _sub_levels():
        v = jnp.where((csub & m) == 0, v, _roll(v, m, 0))
    return v


def _lane_fold(t):
    w = t.shape[1]
    while w > 1:
        w //= 2
        t = t[:, :w] + t[:, w:2 * w]
    return t


def _group_reduce_sum_exact(x, csub):
    s = jnp.sum(x, axis=1, keepdims=True)
    for m in _sub_levels():
        s = s + _roll(s, -m, 0)
    return _spread_from_c0(s, csub)


def _group_reduce_max(x, csub):
    s = jnp.max(x, axis=1, keepdims=True)
    for m in _sub_levels():
        s = jnp.maximum(s, _roll(s, -m, 0))
    return _spread_from_c0(s, csub)


def _row_total(gated, csub):
    nl = gated.shape[1]
    q = nl // 4
    u = (gated[:, :q] + gated[:, q:2 * q]) + \
        (gated[:, 2 * q:3 * q] + gated[:, 3 * q:])
    for m in _sub_levels():
        u = u + _roll(u, -m, 0)
    t = _lane_fold(u)
    return _spread_from_c0(t, csub[:, :1])


def _bitonic_desc(x, lane, csub):
    rows, nl = x.shape

    def gbit_zero(t):
        if (1 << t) < nl:
            return (lane & (1 << t)) == 0
        return (csub & ((1 << t) // nl)) == 0

    k, tk = 2, 1
    while k <= _S:
        dir_desc = gbit_zero(tk) if k < _S else jnp.bool_(True)
        j, tj = k // 2, tk - 1
        while j >= 1:
            is_lower = gbit_zero(tj)
            if j < nl:
                partner = jnp.where(is_lower, _roll(x, -j, 1), _roll(x, j, 1))
            else:
                m = j // nl
                partner = jnp.where(is_lower, _roll(x, -m, 0), _roll(x, m, 0))
            mx = jnp.maximum(x, partner)
            mn = jnp.minimum(x, partner)
            take_max = jnp.logical_not(jnp.logical_xor(dir_desc, is_lower))
            x = jnp.where(take_max, mx, mn)
            j //= 2
            tj -= 1
        k *= 2
        tk += 1
    return x


def _build_mask(lg, am, gates_ref, keep_ref, xt_ref, cumt_ref):
    rows, nl = lg.shape
    lane = jax.lax.broadcasted_iota(jnp.int32, (rows, nl), 1)
    rowi = jax.lax.broadcasted_iota(jnp.int32, (rows, nl), 0)
    csub = rowi & (_NSUB - 1)
    glob = csub * nl + lane

    gates = jax.nn.sigmoid(lg)
    gates_ref[...] = gates
    act = am != 0
    gated = jnp.where(act, gates, jnp.float32(0.0))

    total = jnp.maximum(_row_total(gated, csub), jnp.float32(1e-12))
    shares = jnp.where(act, gated / total, jnp.float32(0.0))

    srt = _bitonic_desc(shares, lane, csub)

    xt_ref[...] = srt.reshape(rows * (nl // 128), 128).T

    def body(i, acc):
        acc = acc + xt_ref[pl.ds(i, 1), :]
        cumt_ref[pl.ds(i, 1), :] = acc
        return acc

    ncols = rows * (nl // 128)
    tot = jax.lax.fori_loop(0, 128, body,
                            jnp.zeros((1, ncols), jnp.float32))

    nblk = ncols // _B
    clane = jax.lax.broadcasted_iota(jnp.int32, (1, ncols), 1)
    cblk = clane & (nblk - 1)
    s = tot
    for step in range(1, nblk):
        s = jnp.where(cblk == step, s + _roll(s, 1, 1), s)
    ex = jnp.where(cblk == 0, jnp.float32(0.0), _roll(s, 1, 1))

    cum = (cumt_ref[...] + ex).T.reshape(rows, nl)

    k0 = _group_reduce_sum_exact(
        (cum <= jnp.float32(_THRESHOLD)).astype(jnp.float32), csub)
    kk = jnp.maximum(k0, jnp.float32(1.0))

    sel = glob == (kk.astype(jnp.int32) - 1)
    sstar = _group_reduce_max(jnp.where(sel, srt, jnp.float32(-1.0)), csub)
    n_greater = _group_reduce_sum_exact(
        (srt > sstar).astype(jnp.float32), csub)
    m = kk - n_greater

    eq = shares == sstar
    p = eq.astype(jnp.float32)
    d = 1
    while d < _S:
        if d < nl:
            piece1 = jnp.where(lane >= d, _roll(p, d, 1), jnp.float32(0.0))
            piece2 = jnp.where((lane < d) & (csub > 0),
                               _roll(_roll(p, -(nl - d), 1), 1, 0),
                               jnp.float32(0.0))
            p = p + piece1 + piece2
        else:
            ms = d // nl
            p = p + jnp.where(csub >= ms, _roll(p, ms, 0), jnp.float32(0.0))
        d *= 2

    keep = act & ((shares > sstar) | (eq & (p <= m)))
    keep_ref[...] = keep.astype(jnp.int32)


def _fused_kernel(h_ref, w_ref, b_ref, am_ref,
                  logits_ref, gates_ref, keep_ref,
                  lg_ref, xt_ref, cumt_ref):
    i = pl.program_id(0)
    nsteps = pl.num_programs(0)
    lg = jax.lax.dot_general(
        w_ref[...], h_ref[...], (((1,), (1,)), ((), ())),
        preferred_element_type=jnp.float32) + b_ref[...]
    logits_ref[...] = lg[None]
    lg_ref[pl.ds(i, 1), :] = lg

    @pl.when(i == nsteps - 1)
    def _():
        _build_mask(lg_ref[...], am_ref[...],
                    gates_ref, keep_ref, xt_ref, cumt_ref)


def kernel(hidden_states, attention_mask, W, b):
    bb, s, d = hidden_states.shape
    h2d = hidden_states.reshape(bb * s, d)
    rows = bb * _NSUB
    nsteps = bb * s // _MV_BLK

    logits_p, gates_p, keep_p = pl.pallas_call(
        _fused_kernel,
        grid=(nsteps,),
        in_specs=[pl.BlockSpec((_MV_BLK, d), lambda i: (i, 0)),
                  pl.BlockSpec((1, d), lambda i: (0, 0)),
                  pl.BlockSpec((1, 1), lambda i: (0, 0)),
                  pl.BlockSpec((rows, _NL), lambda i: (0, 0))],
        out_specs=[pl.BlockSpec((1, 1, _NL), lambda i: (i, 0, 0)),
                   pl.BlockSpec((rows, _NL), lambda i: (0, 0)),
                   pl.BlockSpec((rows, _NL), lambda i: (0, 0))],
        out_shape=[jax.ShapeDtypeStruct((nsteps, 1, _NL), jnp.float32),
                   jax.ShapeDtypeStruct((rows, _NL), jnp.float32),
                   jax.ShapeDtypeStruct((rows, _NL), jnp.int32)],
        scratch_shapes=[pltpu.VMEM((rows, _NL), jnp.float32),
                        pltpu.VMEM((128, rows * _NL // 128), jnp.float32),
                        pltpu.VMEM((128, rows * _NL // 128), jnp.float32)],
    )(h2d, W.reshape(1, d),
      jnp.asarray(b, jnp.float32).reshape(1, 1),
      attention_mask.reshape(rows, _NL))

    return (logits_p.reshape(bb, s), gates_p.reshape(bb, s),
            keep_p.reshape(bb, s).astype(jnp.bool_))
